# baseline (speedup 1.0000x reference)
import functools

import jax
import jax.numpy as jnp
from jax import lax
from jax.experimental import pallas as pl
from jax.experimental.pallas import tpu as pltpu
from jax.experimental.pallas import tpu_sc as plsc

N = 10000
E = 160000
F = 256

NC = 2
NS = 16
CH = 128
NCHUNK = E // CH
BE = 640
NB = E // BE
BN_ = 1000
NBN = N // BN_
BNS = 640
EPS = 1e-5


def _pack_pair(v):
    lo = lax.bitcast_convert_type(v[:, :F].astype(jnp.bfloat16), jnp.uint16)
    hi = lax.bitcast_convert_type(v[:, F:].astype(jnp.bfloat16), jnp.uint16)
    word = lo.astype(jnp.uint32) | (hi.astype(jnp.uint32) << 16)
    return lax.bitcast_convert_type(word, jnp.int32)


def _unpack_pair(w):
    wu = lax.bitcast_convert_type(w, jnp.uint32)
    lo = lax.bitcast_convert_type(wu.astype(jnp.uint16), jnp.bfloat16)
    hi = lax.bitcast_convert_type((wu >> 16).astype(jnp.uint16), jnp.bfloat16)
    return lo, hi


def _tables_body(x_ref, wd_ref, ws_ref, td_ref, ts_ref):
    xb = x_ref[...].astype(jnp.bfloat16)
    td_ref[...] = _pack_pair(
        lax.dot(xb, wd_ref[...], preferred_element_type=jnp.float32))
    ts_ref[...] = _pack_pair(
        lax.dot(xb, ws_ref[...], preferred_element_type=jnp.float32))


def _pass1_body(gd_ref, gs_ref, e_ref, cf_ref, cn_ref, w2f_ref, w2n_ref,
                p_ref, h_ref, m_ref, ss_ref, s1_ref, s2_ref):
    i = pl.program_id(0)

    @pl.when(i == 0)
    def _init():
        s1_ref[...] = jnp.zeros_like(s1_ref)
        s2_ref[...] = jnp.zeros_like(s2_ref)

    glo_d, ghi_d = _unpack_pair(gd_ref[...])
    glo_s, ghi_s = _unpack_pair(gs_ref[...])
    e16 = e_ref[...].astype(jnp.bfloat16)
    pre_f = ((glo_d + glo_s).astype(jnp.float32)
             + lax.dot(e16, cf_ref[...], preferred_element_type=jnp.float32)
             + p_ref[0:1, :])
    pre_n = ((ghi_d + ghi_s).astype(jnp.float32)
             + lax.dot(e16, cn_ref[...], preferred_element_type=jnp.float32)
             + p_ref[1:2, :])
    af = (pre_f * jax.nn.sigmoid(pre_f)).astype(jnp.bfloat16)
    an = (pre_n * jax.nn.sigmoid(pre_n)).astype(jnp.bfloat16)
    h = lax.dot(af, w2f_ref[...], preferred_element_type=jnp.float32) + p_ref[2:3, :]
    m = lax.dot(an, w2n_ref[...], preferred_element_type=jnp.float32) + p_ref[3:4, :]

    s1_ref[...] += jnp.sum(h, axis=0, keepdims=True)
    s2_ref[...] += jnp.sum(h * h, axis=0, keepdims=True)
    h_ref[...] = h.astype(jnp.bfloat16)
    m_ref[...] = m.astype(jnp.bfloat16)

    @pl.when(i == NB - 1)
    def _fin():
        mean = s1_ref[...] / E
        var = s2_ref[...] / E - mean * mean
        scale = p_ref[4:5, :] * lax.rsqrt(var + EPS)
        shift = p_ref[5:6, :] - mean * scale
        ss_ref[...] = jnp.concatenate([scale, shift], axis=0)


def _pass2_body(h_ref, m_ref, ss_ref, msg_ref):
    h = h_ref[...].astype(jnp.float32)
    score = jax.nn.sigmoid(h * ss_ref[0:1, :] + ss_ref[1:2, :])
    msg_ref[...] = score * m_ref[...].astype(jnp.float32)


def _node_bn_body(x_ref, o_ref, gb_ref, out_ref, s1_ref, s2_ref, st_ref):
    i = pl.program_id(0)

    @pl.when(i == 0)
    def _init():
        s1_ref[...] = jnp.zeros_like(s1_ref)
        s2_ref[...] = jnp.zeros_like(s2_ref)

    @pl.when(i < NBN)
    def _stats():
        o = o_ref[...]
        s1_ref[...] += jnp.sum(o, axis=0, keepdims=True)
        s2_ref[...] += jnp.sum(o * o, axis=0, keepdims=True)
        out_ref[...] = o

    @pl.when(i == NBN - 1)
    def _fin():
        mean = s1_ref[...] / N
        var = s2_ref[...] / N - mean * mean
        scale = gb_ref[0:1, :] * lax.rsqrt(var + EPS)
        shift = gb_ref[1:2, :] - mean * scale
        st_ref[...] = jnp.concatenate([scale, shift], axis=0)

    @pl.when(i >= NBN)
    def _apply():
        y = (x_ref[...] + o_ref[...] * st_ref[0:1, :] + st_ref[1:2, :])
        out_ref[...] = jnp.maximum(y, 0.0)


_MESH = plsc.VectorSubcoreMesh(core_axis_name="c", subcore_axis_name="s")
_NW = NC * NS
_KPW = -(-NCHUNK // _NW)
_SROWS = 80
NPAD = 10240
_HALF = NPAD // NC
_HPT = _HALF // NS
_RPT = NPAD // NS


_GROWS = 40


@functools.partial(
    pl.kernel,
    mesh=_MESH,
    out_type=[
        jax.ShapeDtypeStruct((E, F), jnp.int32),
        jax.ShapeDtypeStruct((E, F), jnp.int32),
    ],
    scratch_types=[
        pltpu.VMEM((_GROWS, CH), jnp.int32),
        pltpu.VMEM((_GROWS, CH), jnp.int32),
        pltpu.VMEM((CH, F), jnp.int32),
        pltpu.VMEM((CH, F), jnp.int32),
        pltpu.SemaphoreType.DMA,
        pltpu.SemaphoreType.DMA,
    ],
)
def _sc_gather(td_hbm, ts_hbm, dst_hbm, src_hbm, gd_hbm, gs_hbm,
               idx_d, idx_s, buf0, buf1, sem_g, sem_st):
    wid = lax.axis_index("s") * NC + lax.axis_index("c")
    start = wid * _GROWS
    rows = jnp.where(wid < _NW - 1, _GROWS, NCHUNK - (_NW - 1) * _GROWS)

    pltpu.sync_copy(dst_hbm.at[pl.ds(start, _GROWS)], idx_d)
    pltpu.sync_copy(src_hbm.at[pl.ds(start, _GROWS)], idx_s)

    def run_pass(tab_hbm, idx_all, out_hbm):
        bufs = (buf0, buf1)

        def unit(k, buf):
            pltpu.make_async_copy(tab_hbm.at[idx_all.at[0]], buf,
                                  sem_g).wait()

            @pl.when(k >= 1)
            def _():
                pltpu.make_async_copy(buf, out_hbm.at[pl.ds(0, CH)],
                                      sem_st).wait()

            other = bufs[1] if buf is bufs[0] else bufs[0]

            @pl.when(k + 1 < rows)
            def _():
                pltpu.async_copy(tab_hbm.at[idx_all.at[k + 1]], other, sem_g)

            pltpu.async_copy(
                buf, out_hbm.at[pl.ds((start + k) * CH, CH)], sem_st)

        pltpu.async_copy(tab_hbm.at[idx_all.at[0]], buf0, sem_g)

        def pair(k2, carry):
            a = 2 * k2

            @pl.when(a < rows)
            def _():
                unit(a, buf0)

            @pl.when(a + 1 < rows)
            def _():
                unit(a + 1, buf1)

            return carry

        lax.fori_loop(0, _GROWS // 2, pair, 0)
        pltpu.make_async_copy(buf0, out_hbm.at[pl.ds(0, CH)], sem_st).wait()

    run_pass(td_hbm, idx_d, gd_hbm)
    run_pass(ts_hbm, idx_s, gs_hbm)


@functools.partial(
    pl.kernel,
    mesh=_MESH,
    out_type=jax.ShapeDtypeStruct((NPAD, F), jnp.float32),
    scratch_types=[
        pltpu.VMEM((_SROWS, CH), jnp.int32),
        pltpu.VMEM((_SROWS, CH), jnp.int32),
        pltpu.VMEM((CH, F // NC), jnp.float32),
        pltpu.VMEM((CH, F // NC), jnp.float32),
        pltpu.VMEM_SHARED((_HALF + 8, F // NC), jnp.float32),
        pltpu.SemaphoreType.DMA,
        pltpu.SemaphoreType.DMA,
    ],
)
def _sc_scatter(msg_hbm, dst_hbm, zeros_hbm, out_hbm,
                idx_v, idx2_v, chunk0, chunk1, acc, sem_ld, sem_add):
    c = lax.axis_index("c")
    s = lax.axis_index("s")
    hw = F // NC
    rbase = c * _HALF
    start = s * _SROWS
    rows = jnp.where(s < NS - 1, _SROWS, NCHUNK - (NS - 1) * _SROWS)

    pltpu.sync_copy(dst_hbm.at[pl.ds(start, _SROWS)], idx_v)

    def remap(k, carry):
        @pl.when(k < rows)
        def _():
            for j in range(CH // 16):
                v = idx_v[k, pl.ds(j * 16, 16)] - rbase
                ok = (v >= 0) & (v < _HALF)
                idx2_v[k, pl.ds(j * 16, 16)] = jnp.where(ok, v, _HALF)
        return carry

    lax.fori_loop(0, _SROWS, remap, 0)

    for p in range(2):
        col = p * hw

        pltpu.sync_copy(zeros_hbm, acc.at[pl.ds(s * _HPT, _HPT)])

        @pl.when(s == 0)
        def _zdump():
            pltpu.sync_copy(zeros_hbm.at[pl.ds(0, 8)],
                            acc.at[pl.ds(_HALF, 8)])

        plsc.subcore_barrier()

        chunks = (chunk0, chunk1)

        def unit(k, buf):
            pltpu.make_async_copy(
                msg_hbm.at[pl.ds(0, CH), pl.ds(col, hw)], buf,
                sem_ld).wait()

            @pl.when(k >= 1)
            def _():
                pltpu.make_async_copy(buf, acc.at[idx2_v.at[0]],
                                      sem_add).wait()

            other = chunks[1] if buf is chunks[0] else chunks[0]

            @pl.when(k + 1 < rows)
            def _():
                pltpu.async_copy(
                    msg_hbm.at[pl.ds((start + k + 1) * CH, CH),
                               pl.ds(col, hw)],
                    other, sem_ld)

            pltpu.async_copy(buf, acc.at[idx2_v.at[k]], sem_add, add=True)

        pltpu.async_copy(
            msg_hbm.at[pl.ds(start * CH, CH), pl.ds(col, hw)],
            chunk0, sem_ld)

        def pair(k2, carry):
            a = 2 * k2

            @pl.when(a < rows)
            def _():
                unit(a, chunk0)

            @pl.when(a + 1 < rows)
            def _():
                unit(a + 1, chunk1)

            return carry

        lax.fori_loop(0, _SROWS // 2, pair, 0)
        pltpu.make_async_copy(chunk0, acc.at[idx2_v.at[0]], sem_add).wait()
        plsc.subcore_barrier()

        pltpu.sync_copy(
            acc.at[pl.ds(s * _HPT, _HPT)],
            out_hbm.at[pl.ds(rbase + s * _HPT, _HPT), pl.ds(col, hw)])
        plsc.subcore_barrier()


def kernel(x, edge_index, edge_attr, W1f, b1f, W2f, b2f, g_int, be_int,
           W1n, b1n, W2n, b2n, g_bn, be_bn):
    f32 = jnp.float32
    bf16 = jnp.bfloat16

    pad = jnp.zeros((_GROWS * _NW - NCHUNK, CH), jnp.int32)
    src = jnp.concatenate([edge_index[0].reshape(NCHUNK, CH), pad], axis=0)
    dst = jnp.concatenate([edge_index[1].reshape(NCHUNK, CH), pad], axis=0)

    Wd = jnp.concatenate([W1f[:, :F].T, W1n[:, :F].T], axis=1).astype(bf16)
    Ws = jnp.concatenate([W1f[:, F:2 * F].T, W1n[:, F:2 * F].T], axis=1).astype(bf16)
    CfT = W1f[:, 2 * F:].T.astype(bf16)
    CnT = W1n[:, 2 * F:].T.astype(bf16)
    W2fT = W2f.T.astype(bf16)
    W2nT = W2n.T.astype(bf16)
    zr = jnp.zeros((F,), f32)
    params = jnp.stack([b1f, b1n, b2f, b2n, g_int, be_int, zr, zr])
    gb = jnp.stack([g_bn, be_bn])
    zeros_cb = jnp.zeros((_HPT, F // NC), f32)

    td, ts = pl.pallas_call(
        _tables_body,
        grid=(NBN,),
        in_specs=[
            pl.BlockSpec((BN_, F), lambda i: (i, 0)),
            pl.BlockSpec((F, 2 * F), lambda i: (0, 0)),
            pl.BlockSpec((F, 2 * F), lambda i: (0, 0)),
        ],
        out_specs=[
            pl.BlockSpec((BN_, F), lambda i: (i, 0)),
            pl.BlockSpec((BN_, F), lambda i: (i, 0)),
        ],
        out_shape=[
            jax.ShapeDtypeStruct((N, F), jnp.int32),
            jax.ShapeDtypeStruct((N, F), jnp.int32),
        ],
    )(x, Wd, Ws)

    gd, gs = _sc_gather(td, ts, dst, src)

    h16, m16, ss = pl.pallas_call(
        _pass1_body,
        grid=(NB,),
        in_specs=[
            pl.BlockSpec((BE, F), lambda i: (i, 0)),
            pl.BlockSpec((BE, F), lambda i: (i, 0)),
            pl.BlockSpec((BE, F), lambda i: (i, 0)),
            pl.BlockSpec((F, F), lambda i: (0, 0)),
            pl.BlockSpec((F, F), lambda i: (0, 0)),
            pl.BlockSpec((F, F), lambda i: (0, 0)),
            pl.BlockSpec((F, F), lambda i: (0, 0)),
            pl.BlockSpec((8, F), lambda i: (0, 0)),
        ],
        out_specs=[
            pl.BlockSpec((BE, F), lambda i: (i, 0)),
            pl.BlockSpec((BE, F), lambda i: (i, 0)),
            pl.BlockSpec((2, F), lambda i: (0, 0)),
        ],
        out_shape=[
            jax.ShapeDtypeStruct((E, F), bf16),
            jax.ShapeDtypeStruct((E, F), bf16),
            jax.ShapeDtypeStruct((2, F), f32),
        ],
        scratch_shapes=[
            pltpu.VMEM((1, F), f32),
            pltpu.VMEM((1, F), f32),
        ],
        compiler_params=pltpu.CompilerParams(
            dimension_semantics=("arbitrary",)),
    )(gd, gs, edge_attr, CfT, CnT, W2fT, W2nT, params)

    msg = pl.pallas_call(
        _pass2_body,
        grid=(NB,),
        in_specs=[
            pl.BlockSpec((BE, F), lambda i: (i, 0)),
            pl.BlockSpec((BE, F), lambda i: (i, 0)),
            pl.BlockSpec((2, F), lambda i: (0, 0)),
        ],
        out_specs=pl.BlockSpec((BE, F), lambda i: (i, 0)),
        out_shape=jax.ShapeDtypeStruct((E, F), f32),
    )(h16, m16, ss)

    seg = _sc_scatter(msg, dst, zeros_cb)

    res = pl.pallas_call(
        _node_bn_body,
        grid=(2 * NBN,),
        in_specs=[
            pl.BlockSpec((BN_, F), lambda i: (i % NBN, 0)),
            pl.BlockSpec((BN_, F), lambda i: (i % NBN, 0)),
            pl.BlockSpec((2, F), lambda i: (0, 0)),
        ],
        out_specs=pl.BlockSpec((BN_, F), lambda i: (i % NBN, 0)),
        out_shape=jax.ShapeDtypeStruct((N, F), f32),
        scratch_shapes=[
            pltpu.VMEM((1, F), f32),
            pltpu.VMEM((1, F), f32),
            pltpu.VMEM((2, F), f32),
        ],
        compiler_params=pltpu.CompilerParams(
            dimension_semantics=("arbitrary",)),
    )(x, seg, gb)

    return res


# speedup vs baseline: 1.1355x; 1.1355x over previous
import functools

import jax
import jax.numpy as jnp
from jax import lax
from jax.experimental import pallas as pl
from jax.experimental.pallas import tpu as pltpu
from jax.experimental.pallas import tpu_sc as plsc

N = 10000
E = 160000
F = 256

NC = 2
NS = 16
CH = 128
NCHUNK = E // CH
BE = 1280
NB = E // BE
BN_ = 1000
NBN = N // BN_
BNS = 640
EPS = 1e-5


def _pack_pair(v):
    lo = lax.bitcast_convert_type(v[:, :F].astype(jnp.bfloat16), jnp.uint16)
    hi = lax.bitcast_convert_type(v[:, F:].astype(jnp.bfloat16), jnp.uint16)
    word = lo.astype(jnp.uint32) | (hi.astype(jnp.uint32) << 16)
    return lax.bitcast_convert_type(word, jnp.int32)


def _unpack_pair(w):
    wu = lax.bitcast_convert_type(w, jnp.uint32)
    lo = lax.bitcast_convert_type(wu.astype(jnp.uint16), jnp.bfloat16)
    hi = lax.bitcast_convert_type((wu >> 16).astype(jnp.uint16), jnp.bfloat16)
    return lo, hi


def _tables_body(x_ref, wd_ref, ws_ref, td_ref, ts_ref):
    xb = x_ref[...].astype(jnp.bfloat16)
    td_ref[...] = _pack_pair(
        lax.dot(xb, wd_ref[...], preferred_element_type=jnp.float32))
    ts_ref[...] = _pack_pair(
        lax.dot(xb, ws_ref[...], preferred_element_type=jnp.float32))


def _pass1_body(gd_ref, gs_ref, e_ref, cf_ref, cn_ref, w2f_ref, w2n_ref,
                p_ref, h_ref, m_ref, ss_ref, s1_ref, s2_ref):
    i = pl.program_id(0)

    @pl.when(i == 0)
    def _init():
        s1_ref[...] = jnp.zeros_like(s1_ref)
        s2_ref[...] = jnp.zeros_like(s2_ref)

    glo_d, ghi_d = _unpack_pair(gd_ref[...])
    glo_s, ghi_s = _unpack_pair(gs_ref[...])
    e16 = e_ref[...].astype(jnp.bfloat16)
    pre_f = ((glo_d + glo_s).astype(jnp.float32)
             + lax.dot(e16, cf_ref[...], preferred_element_type=jnp.float32)
             + p_ref[0:1, :])
    pre_n = ((ghi_d + ghi_s).astype(jnp.float32)
             + lax.dot(e16, cn_ref[...], preferred_element_type=jnp.float32)
             + p_ref[1:2, :])
    af = (pre_f * jax.nn.sigmoid(pre_f)).astype(jnp.bfloat16)
    an = (pre_n * jax.nn.sigmoid(pre_n)).astype(jnp.bfloat16)
    h = lax.dot(af, w2f_ref[...], preferred_element_type=jnp.float32) + p_ref[2:3, :]
    m = lax.dot(an, w2n_ref[...], preferred_element_type=jnp.float32) + p_ref[3:4, :]

    s1_ref[...] += jnp.sum(h, axis=0, keepdims=True)
    s2_ref[...] += jnp.sum(h * h, axis=0, keepdims=True)
    h_ref[...] = h.astype(jnp.bfloat16)
    m_ref[...] = m.astype(jnp.bfloat16)

    @pl.when(i == NB - 1)
    def _fin():
        mean = s1_ref[...] / E
        var = s2_ref[...] / E - mean * mean
        scale = p_ref[4:5, :] * lax.rsqrt(var + EPS)
        shift = p_ref[5:6, :] - mean * scale
        ss_ref[...] = jnp.concatenate([scale, shift], axis=0)


def _pass2_body(h_ref, m_ref, ss_ref, msg_ref):
    h = h_ref[...].astype(jnp.float32)
    score = jax.nn.sigmoid(h * ss_ref[0:1, :] + ss_ref[1:2, :])
    msg_ref[...] = score * m_ref[...].astype(jnp.float32)


def _node_bn_body(x_ref, o_ref, gb_ref, out_ref, s1_ref, s2_ref, st_ref):
    i = pl.program_id(0)

    @pl.when(i == 0)
    def _init():
        s1_ref[...] = jnp.zeros_like(s1_ref)
        s2_ref[...] = jnp.zeros_like(s2_ref)

    @pl.when(i < NBN)
    def _stats():
        o = o_ref[...]
        s1_ref[...] += jnp.sum(o, axis=0, keepdims=True)
        s2_ref[...] += jnp.sum(o * o, axis=0, keepdims=True)
        out_ref[...] = o

    @pl.when(i == NBN - 1)
    def _fin():
        mean = s1_ref[...] / N
        var = s2_ref[...] / N - mean * mean
        scale = gb_ref[0:1, :] * lax.rsqrt(var + EPS)
        shift = gb_ref[1:2, :] - mean * scale
        st_ref[...] = jnp.concatenate([scale, shift], axis=0)

    @pl.when(i >= NBN)
    def _apply():
        y = (x_ref[...] + o_ref[...] * st_ref[0:1, :] + st_ref[1:2, :])
        out_ref[...] = jnp.maximum(y, 0.0)


_MESH = plsc.VectorSubcoreMesh(core_axis_name="c", subcore_axis_name="s")
_NW = NC * NS
_KPW = -(-NCHUNK // _NW)
_SROWS = 80
NPAD = 10240
_HALF = NPAD // NC
_HPT = _HALF // NS
_RPT = NPAD // NS


_GROWS = 40


@functools.partial(
    pl.kernel,
    mesh=_MESH,
    out_type=[
        jax.ShapeDtypeStruct((E, F), jnp.int32),
        jax.ShapeDtypeStruct((E, F), jnp.int32),
    ],
    scratch_types=[
        pltpu.VMEM((_GROWS, CH), jnp.int32),
        pltpu.VMEM((_GROWS, CH), jnp.int32),
        pltpu.VMEM((CH, F), jnp.int32),
        pltpu.VMEM((CH, F), jnp.int32),
        pltpu.SemaphoreType.DMA,
        pltpu.SemaphoreType.DMA,
    ],
)
def _sc_gather(td_hbm, ts_hbm, dst_hbm, src_hbm, gd_hbm, gs_hbm,
               idx_d, idx_s, buf0, buf1, sem_g, sem_st):
    wid = lax.axis_index("s") * NC + lax.axis_index("c")
    start = wid * _GROWS
    rows = jnp.where(wid < _NW - 1, _GROWS, NCHUNK - (_NW - 1) * _GROWS)

    pltpu.sync_copy(dst_hbm.at[pl.ds(start, _GROWS)], idx_d)
    pltpu.sync_copy(src_hbm.at[pl.ds(start, _GROWS)], idx_s)

    def run_pass(tab_hbm, idx_all, out_hbm):
        bufs = (buf0, buf1)

        def unit(k, buf):
            pltpu.make_async_copy(tab_hbm.at[idx_all.at[0]], buf,
                                  sem_g).wait()

            @pl.when(k >= 1)
            def _():
                pltpu.make_async_copy(buf, out_hbm.at[pl.ds(0, CH)],
                                      sem_st).wait()

            other = bufs[1] if buf is bufs[0] else bufs[0]

            @pl.when(k + 1 < rows)
            def _():
                pltpu.async_copy(tab_hbm.at[idx_all.at[k + 1]], other, sem_g)

            pltpu.async_copy(
                buf, out_hbm.at[pl.ds((start + k) * CH, CH)], sem_st)

        pltpu.async_copy(tab_hbm.at[idx_all.at[0]], buf0, sem_g)

        def pair(k2, carry):
            a = 2 * k2

            @pl.when(a < rows)
            def _():
                unit(a, buf0)

            @pl.when(a + 1 < rows)
            def _():
                unit(a + 1, buf1)

            return carry

        lax.fori_loop(0, _GROWS // 2, pair, 0)
        pltpu.make_async_copy(buf0, out_hbm.at[pl.ds(0, CH)], sem_st).wait()

    run_pass(td_hbm, idx_d, gd_hbm)
    run_pass(ts_hbm, idx_s, gs_hbm)


@functools.partial(
    pl.kernel,
    mesh=_MESH,
    out_type=jax.ShapeDtypeStruct((NPAD, F), jnp.float32),
    scratch_types=[
        pltpu.VMEM((_SROWS, CH), jnp.int32),
        pltpu.VMEM((_SROWS, CH), jnp.int32),
        pltpu.VMEM((CH, F // NC), jnp.float32),
        pltpu.VMEM((CH, F // NC), jnp.float32),
        pltpu.VMEM_SHARED((_HALF + 8, F // NC), jnp.float32),
        pltpu.SemaphoreType.DMA,
        pltpu.SemaphoreType.DMA,
    ],
)
def _sc_scatter(msg_hbm, dst_hbm, zeros_hbm, out_hbm,
                idx_v, idx2_v, chunk0, chunk1, acc, sem_ld, sem_add):
    c = lax.axis_index("c")
    s = lax.axis_index("s")
    hw = F // NC
    rbase = c * _HALF
    start = s * _SROWS
    rows = jnp.where(s < NS - 1, _SROWS, NCHUNK - (NS - 1) * _SROWS)

    pltpu.sync_copy(dst_hbm.at[pl.ds(start, _SROWS)], idx_v)

    def remap(k, carry):
        @pl.when(k < rows)
        def _():
            for j in range(CH // 16):
                v = idx_v[k, pl.ds(j * 16, 16)] - rbase
                ok = (v >= 0) & (v < _HALF)
                idx2_v[k, pl.ds(j * 16, 16)] = jnp.where(ok, v, _HALF)
        return carry

    lax.fori_loop(0, _SROWS, remap, 0)

    for p in range(2):
        col = p * hw

        pltpu.sync_copy(zeros_hbm, acc.at[pl.ds(s * _HPT, _HPT)])

        @pl.when(s == 0)
        def _zdump():
            pltpu.sync_copy(zeros_hbm.at[pl.ds(0, 8)],
                            acc.at[pl.ds(_HALF, 8)])

        plsc.subcore_barrier()

        chunks = (chunk0, chunk1)

        def unit(k, buf):
            pltpu.make_async_copy(
                msg_hbm.at[pl.ds(0, CH), pl.ds(col, hw)], buf,
                sem_ld).wait()

            @pl.when(k >= 1)
            def _():
                pltpu.make_async_copy(buf, acc.at[idx2_v.at[0]],
                                      sem_add).wait()

            other = chunks[1] if buf is chunks[0] else chunks[0]

            @pl.when(k + 1 < rows)
            def _():
                pltpu.async_copy(
                    msg_hbm.at[pl.ds((start + k + 1) * CH, CH),
                               pl.ds(col, hw)],
                    other, sem_ld)

            pltpu.async_copy(buf, acc.at[idx2_v.at[k]], sem_add, add=True)

        pltpu.async_copy(
            msg_hbm.at[pl.ds(start * CH, CH), pl.ds(col, hw)],
            chunk0, sem_ld)

        def pair(k2, carry):
            a = 2 * k2

            @pl.when(a < rows)
            def _():
                unit(a, chunk0)

            @pl.when(a + 1 < rows)
            def _():
                unit(a + 1, chunk1)

            return carry

        lax.fori_loop(0, _SROWS // 2, pair, 0)
        pltpu.make_async_copy(chunk0, acc.at[idx2_v.at[0]], sem_add).wait()
        plsc.subcore_barrier()

        pltpu.sync_copy(
            acc.at[pl.ds(s * _HPT, _HPT)],
            out_hbm.at[pl.ds(rbase + s * _HPT, _HPT), pl.ds(col, hw)])
        plsc.subcore_barrier()


def kernel(x, edge_index, edge_attr, W1f, b1f, W2f, b2f, g_int, be_int,
           W1n, b1n, W2n, b2n, g_bn, be_bn):
    f32 = jnp.float32
    bf16 = jnp.bfloat16

    pad = jnp.zeros((_GROWS * _NW - NCHUNK, CH), jnp.int32)
    src = jnp.concatenate([edge_index[0].reshape(NCHUNK, CH), pad], axis=0)
    dst = jnp.concatenate([edge_index[1].reshape(NCHUNK, CH), pad], axis=0)

    Wd = jnp.concatenate([W1f[:, :F].T, W1n[:, :F].T], axis=1).astype(bf16)
    Ws = jnp.concatenate([W1f[:, F:2 * F].T, W1n[:, F:2 * F].T], axis=1).astype(bf16)
    CfT = W1f[:, 2 * F:].T.astype(bf16)
    CnT = W1n[:, 2 * F:].T.astype(bf16)
    W2fT = W2f.T.astype(bf16)
    W2nT = W2n.T.astype(bf16)
    zr = jnp.zeros((F,), f32)
    params = jnp.stack([b1f, b1n, b2f, b2n, g_int, be_int, zr, zr])
    gb = jnp.stack([g_bn, be_bn])
    zeros_cb = jnp.zeros((_HPT, F // NC), f32)

    td, ts = pl.pallas_call(
        _tables_body,
        grid=(NBN,),
        in_specs=[
            pl.BlockSpec((BN_, F), lambda i: (i, 0)),
            pl.BlockSpec((F, 2 * F), lambda i: (0, 0)),
            pl.BlockSpec((F, 2 * F), lambda i: (0, 0)),
        ],
        out_specs=[
            pl.BlockSpec((BN_, F), lambda i: (i, 0)),
            pl.BlockSpec((BN_, F), lambda i: (i, 0)),
        ],
        out_shape=[
            jax.ShapeDtypeStruct((N, F), jnp.int32),
            jax.ShapeDtypeStruct((N, F), jnp.int32),
        ],
    )(x, Wd, Ws)

    gd, gs = _sc_gather(td, ts, dst, src)

    h16, m16, ss = pl.pallas_call(
        _pass1_body,
        grid=(NB,),
        in_specs=[
            pl.BlockSpec((BE, F), lambda i: (i, 0)),
            pl.BlockSpec((BE, F), lambda i: (i, 0)),
            pl.BlockSpec((BE, F), lambda i: (i, 0)),
            pl.BlockSpec((F, F), lambda i: (0, 0)),
            pl.BlockSpec((F, F), lambda i: (0, 0)),
            pl.BlockSpec((F, F), lambda i: (0, 0)),
            pl.BlockSpec((F, F), lambda i: (0, 0)),
            pl.BlockSpec((8, F), lambda i: (0, 0)),
        ],
        out_specs=[
            pl.BlockSpec((BE, F), lambda i: (i, 0)),
            pl.BlockSpec((BE, F), lambda i: (i, 0)),
            pl.BlockSpec((2, F), lambda i: (0, 0)),
        ],
        out_shape=[
            jax.ShapeDtypeStruct((E, F), bf16),
            jax.ShapeDtypeStruct((E, F), bf16),
            jax.ShapeDtypeStruct((2, F), f32),
        ],
        scratch_shapes=[
            pltpu.VMEM((1, F), f32),
            pltpu.VMEM((1, F), f32),
        ],
        compiler_params=pltpu.CompilerParams(
            dimension_semantics=("arbitrary",)),
    )(gd, gs, edge_attr, CfT, CnT, W2fT, W2nT, params)

    msg = pl.pallas_call(
        _pass2_body,
        grid=(NB,),
        in_specs=[
            pl.BlockSpec((BE, F), lambda i: (i, 0)),
            pl.BlockSpec((BE, F), lambda i: (i, 0)),
            pl.BlockSpec((2, F), lambda i: (0, 0)),
        ],
        out_specs=pl.BlockSpec((BE, F), lambda i: (i, 0)),
        out_shape=jax.ShapeDtypeStruct((E, F), f32),
    )(h16, m16, ss)

    seg = _sc_scatter(msg, dst, zeros_cb)

    res = pl.pallas_call(
        _node_bn_body,
        grid=(2 * NBN,),
        in_specs=[
            pl.BlockSpec((BN_, F), lambda i: (i % NBN, 0)),
            pl.BlockSpec((BN_, F), lambda i: (i % NBN, 0)),
            pl.BlockSpec((2, F), lambda i: (0, 0)),
        ],
        out_specs=pl.BlockSpec((BN_, F), lambda i: (i % NBN, 0)),
        out_shape=jax.ShapeDtypeStruct((N, F), f32),
        scratch_shapes=[
            pltpu.VMEM((1, F), f32),
            pltpu.VMEM((1, F), f32),
            pltpu.VMEM((2, F), f32),
        ],
        compiler_params=pltpu.CompilerParams(
            dimension_semantics=("arbitrary",)),
    )(x, seg, gb)

    return res
